# baseline (speedup 1.0000x reference)
import jax
import jax.numpy as jnp
from jax.experimental import pallas as pl
from jax.experimental.pallas import tpu as pltpu

ALPHA = 4


def kernel(frames):
    C, T, H, W = frames.shape
    n_slow = T // ALPHA
    idx = jnp.linspace(0.0, float(T - 1), n_slow).astype(jnp.int32)

    def dma_gather(idx_ref, src, out, buf, in_sems, out_sems):
        def in_copy(j, c):
            return pltpu.make_async_copy(
                src.at[c, idx_ref[j]], buf.at[j, c], in_sems.at[j, c]
            )

        def out_copy(j, c):
            return pltpu.make_async_copy(
                buf.at[j, c], out.at[c, j], out_sems.at[j, c]
            )

        for j in range(n_slow):
            for c in range(C):
                in_copy(j, c).start()
        for j in range(n_slow):
            for c in range(C):
                in_copy(j, c).wait()
                out_copy(j, c).start()
        for j in range(n_slow):
            for c in range(C):
                out_copy(j, c).wait()

    slow = pl.pallas_call(
        dma_gather,
        grid_spec=pltpu.PrefetchScalarGridSpec(
            num_scalar_prefetch=1,
            grid=(),
            in_specs=[pl.BlockSpec(memory_space=pl.ANY)],
            out_specs=pl.BlockSpec(memory_space=pl.ANY),
            scratch_shapes=[
                pltpu.VMEM((n_slow, C, H, W), frames.dtype),
                pltpu.SemaphoreType.DMA((n_slow, C)),
                pltpu.SemaphoreType.DMA((n_slow, C)),
            ],
        ),
        out_shape=jax.ShapeDtypeStruct((C, n_slow, H, W), frames.dtype),
    )(idx, frames)

    return (slow, frames)


# speedup vs baseline: 1.0173x; 1.0173x over previous
import jax
import jax.numpy as jnp
from jax.experimental import pallas as pl
from jax.experimental.pallas import tpu as pltpu

ALPHA = 4


def kernel(frames):
    C, T, H, W = frames.shape
    n_slow = T // ALPHA
    idx = jnp.linspace(0.0, float(T - 1), n_slow).astype(jnp.int32)

    def dma_gather(idx_ref, src, out, buf, in_sems, out_sems):
        def in_copy(j):
            return pltpu.make_async_copy(
                src.at[:, idx_ref[j]], buf.at[j], in_sems.at[j]
            )

        def out_copy(j):
            return pltpu.make_async_copy(buf.at[j], out.at[:, j], out_sems.at[j])

        for j in range(n_slow):
            in_copy(j).start()
        for j in range(n_slow):
            in_copy(j).wait()
            out_copy(j).start()
        for j in range(n_slow):
            out_copy(j).wait()

    slow = pl.pallas_call(
        dma_gather,
        grid_spec=pltpu.PrefetchScalarGridSpec(
            num_scalar_prefetch=1,
            grid=(),
            in_specs=[pl.BlockSpec(memory_space=pl.ANY)],
            out_specs=pl.BlockSpec(memory_space=pl.ANY),
            scratch_shapes=[
                pltpu.VMEM((n_slow, C, H, W), frames.dtype),
                pltpu.SemaphoreType.DMA((n_slow,)),
                pltpu.SemaphoreType.DMA((n_slow,)),
            ],
        ),
        out_shape=jax.ShapeDtypeStruct((C, n_slow, H, W), frames.dtype),
    )(idx, frames)

    return (slow, frames)
